# Initial kernel scaffold; baseline (speedup 1.0000x reference)
import jax
import jax.numpy as jnp
from jax.experimental import pallas as pl


def kernel(x, d, Z_i, Z_j, exp_prefactors, exp_lengthscales, exp_powers):
    raise NotImplementedError("write your pallas kernel here")


# speedup vs baseline: 5.5763x; 5.5763x over previous
import functools

import jax
import jax.numpy as jnp
from jax import lax
from jax.experimental import pallas as pl
from jax.experimental.pallas import tpu as pltpu
from jax.experimental.pallas import tpu_sc as plsc

N = 160000
NUM_ELEMENTS = 100
NUM_FEATURES = 64
M = 4
ROW = M * NUM_FEATURES
NTAB = NUM_ELEMENTS * NUM_ELEMENTS

NW = 32
CHUNK = 128
CPW = 40
NPAD = NW * CHUNK * CPW

_sc_mesh = plsc.VectorSubcoreMesh(core_axis_name="c", subcore_axis_name="s")


@functools.partial(
    pl.kernel,
    out_type=jax.ShapeDtypeStruct((NPAD, ROW), jnp.float32),
    mesh=_sc_mesh,
    scratch_types=[
        pltpu.VMEM((CPW, CHUNK), jnp.int32),
        pltpu.VMEM((CHUNK, ROW), jnp.float32),
        pltpu.SemaphoreType.DMA,
    ],
)
def _sc_gather(tab, idx, out, idx_v, buf, sem):
    wid = lax.axis_index("s") * 2 + lax.axis_index("c")
    pltpu.sync_copy(idx.at[wid], idx_v)

    def body(j, carry):
        row = idx_v.at[j]
        pltpu.async_copy(tab.at[row], buf, sem).wait()
        base = wid * (CPW * CHUNK) + j * CHUNK
        pltpu.sync_copy(buf, out.at[pl.ds(base, CHUNK)])
        return carry

    lax.fori_loop(0, CPW, body, 0)


def _tc_body(x_ref, d_ref, g_ref, o_ref):
    g = g_ref[...]
    p = g[:, 0:NUM_FEATURES]
    l = g[:, NUM_FEATURES:2 * NUM_FEATURES]
    w = g[:, 2 * NUM_FEATURES:3 * NUM_FEATURES]
    d = d_ref[...]
    shift = p * jnp.exp(-jnp.exp(w * jnp.log(d / l)))
    pad = jnp.zeros((shift.shape[0], ROW - NUM_FEATURES), jnp.float32)
    o_ref[...] = x_ref[...] + jnp.concatenate([shift, pad], axis=1)


def kernel(x, d, Z_i, Z_j, exp_prefactors, exp_lengthscales, exp_powers):
    idx = Z_i.astype(jnp.int32) * NUM_ELEMENTS + Z_j.astype(jnp.int32)
    idx = jnp.concatenate([idx, jnp.zeros((NPAD - N,), jnp.int32)])
    idx3 = idx.reshape(NW, CPW, CHUNK)
    tp = exp_prefactors.reshape(NTAB, NUM_FEATURES)
    tl = exp_lengthscales.reshape(NTAB, NUM_FEATURES)
    tw = exp_powers.reshape(NTAB, NUM_FEATURES)
    tab = jnp.concatenate([tp, tl, tw, tw], axis=-1)

    g = _sc_gather(tab, idx3)

    R = 800
    grid = (N // R,)
    out2 = pl.pallas_call(
        _tc_body,
        grid=grid,
        in_specs=[
            pl.BlockSpec((R, ROW), lambda i: (i, 0)),
            pl.BlockSpec((R, 1), lambda i: (i, 0)),
            pl.BlockSpec((R, ROW), lambda i: (i, 0)),
        ],
        out_specs=pl.BlockSpec((R, ROW), lambda i: (i, 0)),
        out_shape=jax.ShapeDtypeStruct((N, ROW), jnp.float32),
    )(x.reshape(N, ROW), d.reshape(N, 1), g)

    return out2.reshape(N, 1, M, NUM_FEATURES)


# speedup vs baseline: 5.9825x; 1.0728x over previous
import functools

import jax
import jax.numpy as jnp
from jax import lax
from jax.experimental import pallas as pl
from jax.experimental.pallas import tpu as pltpu
from jax.experimental.pallas import tpu_sc as plsc

N = 160000
NUM_ELEMENTS = 100
NUM_FEATURES = 64
M = 4
ROW = M * NUM_FEATURES
NTAB = NUM_ELEMENTS * NUM_ELEMENTS

NW = 32
CHUNK = 128
CPW = 40
NPAD = NW * CHUNK * CPW

_sc_mesh = plsc.VectorSubcoreMesh(core_axis_name="c", subcore_axis_name="s")


@functools.partial(
    pl.kernel,
    out_type=jax.ShapeDtypeStruct((NPAD, ROW), jnp.float32),
    mesh=_sc_mesh,
    scratch_types=[
        pltpu.VMEM((CPW, CHUNK), jnp.int32),
        pltpu.VMEM((CHUNK, ROW), jnp.float32),
        pltpu.VMEM((CHUNK, ROW), jnp.float32),
        pltpu.SemaphoreType.DMA,
        pltpu.SemaphoreType.DMA,
    ],
)
def _sc_gather(tab, idx, out, idx_v, buf_a, buf_b, sem_a, sem_b):
    wid = lax.axis_index("s") * 2 + lax.axis_index("c")
    base = wid * (CPW * CHUNK)
    pltpu.sync_copy(idx.at[wid], idx_v)

    pltpu.async_copy(tab.at[idx_v.at[0]], buf_a, sem_a)

    def body(t, carry):
        j0 = 2 * t
        j1 = j0 + 1
        pltpu.async_copy(tab.at[idx_v.at[j1]], buf_b, sem_b)
        pltpu.make_async_copy(tab.at[idx_v.at[j0]], buf_a, sem_a).wait()
        pltpu.sync_copy(buf_a, out.at[pl.ds(base + j0 * CHUNK, CHUNK)])

        @pl.when(t < CPW // 2 - 1)
        def _():
            pltpu.async_copy(tab.at[idx_v.at[j0 + 2]], buf_a, sem_a)

        pltpu.make_async_copy(tab.at[idx_v.at[j1]], buf_b, sem_b).wait()
        pltpu.sync_copy(buf_b, out.at[pl.ds(base + j1 * CHUNK, CHUNK)])
        return carry

    lax.fori_loop(0, CPW // 2, body, 0)


def _tc_body(x_ref, d_ref, g_ref, o_ref):
    g = g_ref[...]
    p = g[:, 0:NUM_FEATURES]
    l = g[:, NUM_FEATURES:2 * NUM_FEATURES]
    w = g[:, 2 * NUM_FEATURES:3 * NUM_FEATURES]
    d = d_ref[...]
    shift = p * jnp.exp(-jnp.exp(w * jnp.log(d / l)))
    pad = jnp.zeros((shift.shape[0], ROW - NUM_FEATURES), jnp.float32)
    o_ref[...] = x_ref[...] + jnp.concatenate([shift, pad], axis=1)


def kernel(x, d, Z_i, Z_j, exp_prefactors, exp_lengthscales, exp_powers):
    idx = Z_i.astype(jnp.int32) * NUM_ELEMENTS + Z_j.astype(jnp.int32)
    idx = jnp.concatenate([idx, jnp.zeros((NPAD - N,), jnp.int32)])
    idx3 = idx.reshape(NW, CPW, CHUNK)
    tp = exp_prefactors.reshape(NTAB, NUM_FEATURES)
    tl = exp_lengthscales.reshape(NTAB, NUM_FEATURES)
    tw = exp_powers.reshape(NTAB, NUM_FEATURES)
    tab = jnp.concatenate([tp, tl, tw, tw], axis=-1)

    g = _sc_gather(tab, idx3)

    R = 800
    grid = (N // R,)
    out2 = pl.pallas_call(
        _tc_body,
        grid=grid,
        in_specs=[
            pl.BlockSpec((R, ROW), lambda i: (i, 0)),
            pl.BlockSpec((R, 1), lambda i: (i, 0)),
            pl.BlockSpec((R, ROW), lambda i: (i, 0)),
        ],
        out_specs=pl.BlockSpec((R, ROW), lambda i: (i, 0)),
        out_shape=jax.ShapeDtypeStruct((N, ROW), jnp.float32),
    )(x.reshape(N, ROW), d.reshape(N, 1), g)

    return out2.reshape(N, 1, M, NUM_FEATURES)
